# baseline (speedup 1.0000x reference)
import functools

import jax
import jax.numpy as jnp
from jax import lax
from jax.experimental import pallas as pl
from jax.experimental.pallas import tpu as pltpu
from jax.experimental.pallas import tpu_sc as plsc

B = 4096
H = 50
D = 64
LS = 20
SEGS = ((0, 2), (2, 9), (9, 20))

NC = 2
NS = 16
NW = NC * NS
RPW = B // NW
NBUF = 2


def _sc_pool_kernel(table_hbm, idx_hbm, off_hbm, out_hbm, idx_v, off_v,
                    buf0, buf1, acc, sem0, sem1):
    c = lax.axis_index("c")
    s = lax.axis_index("s")
    wid = s * NC + c
    base = wid * RPW

    bufs = (buf0, buf1)
    sems = (sem0, sem1)

    pltpu.sync_copy(idx_hbm.at[wid], idx_v)
    pltpu.sync_copy(off_hbm.at[wid], off_v)

    def gather(j, slot):
        return pltpu.make_async_copy(
            table_hbm.at[idx_v.at[j]], bufs[slot], sems[slot])

    gather(0, 0).start()
    gather(1, 1).start()

    def zero_body(r, _):
        z = jnp.zeros((16,), jnp.float32)
        for d in range(4):
            acc[r, pl.ds(d * 16, 16)] = z
        return 0
    lax.fori_loop(0, RPW, zero_body, 0)

    def accum(j, slot):
        buf = bufs[slot]

        def grp_body(g, _):
            offs = off_v[g, j]
            for rr in range(16):
                r = g * 16 + rr
                lo_sel = offs[rr] == 0
                for d in range(4):
                    lo = buf[r, pl.ds(d * 16, 16)]
                    hi = buf[r, pl.ds(D + d * 16, 16)]
                    plsc.addupdate(acc.at[r, pl.ds(d * 16, 16)],
                                   jnp.where(lo_sel, lo, hi))
            return 0
        lax.fori_loop(0, RPW // 16, grp_body, 0)

    def main_body(jj, _):
        for slot in range(NBUF):
            j = jj * NBUF + slot
            gather(j, slot).wait()
            accum(j, slot)

            @pl.when(j + NBUF < H)
            def _():
                gather(j + NBUF, slot).start()
        return 0

    lax.fori_loop(0, H // NBUF, main_body, 0)

    pltpu.sync_copy(acc, out_hbm.at[pl.ds(base, RPW)])


@functools.lru_cache(maxsize=None)
def _sc_pool():
    return pl.kernel(
        _sc_pool_kernel,
        out_type=jax.ShapeDtypeStruct((B, D), jnp.float32),
        mesh=plsc.VectorSubcoreMesh(core_axis_name="c", subcore_axis_name="s",
                                    num_cores=NC, num_subcores=NS),
        scratch_types=[
            pltpu.VMEM((H, RPW), jnp.int32),
            pltpu.VMEM((RPW // 16, H, 16), jnp.int32),
            pltpu.VMEM((RPW, 128), jnp.float32),
            pltpu.VMEM((RPW, 128), jnp.float32),
            pltpu.VMEM((RPW, D), jnp.float32),
            pltpu.SemaphoreType.DMA,
            pltpu.SemaphoreType.DMA,
        ],
    )


_PAD_BLK = 8192
_FOLD = 62 * _PAD_BLK


def _tc_pack_kernel(lo_ref, hi_ref, out_ref):
    out_ref[...] = jnp.concatenate([lo_ref[...].T, hi_ref[...].T], axis=1)


def _tc_pack(emb_t):
    return pl.pallas_call(
        _tc_pack_kernel,
        grid=(_FOLD // _PAD_BLK,),
        in_specs=[
            pl.BlockSpec((D, _PAD_BLK), lambda i: (0, i)),
            pl.BlockSpec((D, _PAD_BLK),
                         lambda i: (0, jnp.minimum(i, 60) + _FOLD // _PAD_BLK)),
        ],
        out_specs=pl.BlockSpec((_PAD_BLK, 2 * D), lambda i: (i, 0)),
        out_shape=jax.ShapeDtypeStruct((_FOLD, 2 * D), jnp.float32),
    )(emb_t, emb_t)


def _tc_head_kernel(us_ref, mask_ref, y_ref, ob_ref, w_ref,
                    logit_ref, loss_ref):
    x_len = jnp.sum(mask_ref[...], axis=1, keepdims=True)
    user_rep = us_ref[...] / x_len
    wu = lax.dot_general(user_rep, w_ref[...], (((1,), (1,)), ((), ())),
                         preferred_element_type=jnp.float32)
    y = y_ref[...]
    ob = ob_ref[...]
    col = lax.broadcasted_iota(jnp.int32, (1, LS), 1)
    loss = jnp.float32(0.0)
    logit = jnp.zeros((B, LS), jnp.float32)
    for (s, e) in SEGS:
        m = (col >= s) & (col < e)
        wc = jnp.where(m, wu * ob, 0.0)
        row_sum = jnp.sum(wc, axis=1)
        row_mask = (row_sum != 0.0).astype(jnp.float32)
        cnt = jnp.sum(row_mask)
        denom = jnp.sum(jnp.where(m, jnp.exp(wc), 0.0), axis=1)
        dot_y = jnp.sum(wc * y, axis=1)
        nll = jnp.sum(row_mask * (jnp.log(denom) - dot_y))
        loss = loss + jnp.where(cnt > 0, nll / cnt, jnp.float32(0.0))
        mx = jnp.max(jnp.where(m, wu, -1e30), axis=1, keepdims=True)
        ex = jnp.where(m, jnp.exp(wu - mx), 0.0)
        sm = ex / jnp.sum(ex, axis=1, keepdims=True)
        logit = jnp.where(m, sm, logit)
    logit_ref[...] = logit
    loss_ref[...] = jnp.broadcast_to(loss, (1, 1))


def _tc_head(user_sum, x_mask, y, ob, W):
    return pl.pallas_call(
        _tc_head_kernel,
        out_shape=(
            jax.ShapeDtypeStruct((B, LS), jnp.float32),
            jax.ShapeDtypeStruct((1, 1), jnp.float32),
        ),
    )(user_sum, x_mask, y, ob, W)


def kernel(x, x_mask, y, ob, item_emb, W):
    idx3 = x.reshape(NW, RPW, H).transpose(0, 2, 1)
    pidx3 = jnp.where(idx3 < _FOLD, idx3, idx3 - _FOLD)
    poff3 = jnp.where(idx3 < _FOLD, 0, D).astype(jnp.int32)
    poff4 = poff3.reshape(NW, H, RPW // 16, 16).transpose(0, 2, 1, 3)
    table2 = _tc_pack(item_emb.T)
    user_sum = _sc_pool()(table2, pidx3, poff4)
    logit, loss = _tc_head(user_sum, x_mask, y, ob, W)
    return (logit, loss[0, 0])


# speedup vs baseline: 1.1268x; 1.1268x over previous
import functools

import jax
import jax.numpy as jnp
from jax import lax
from jax.experimental import pallas as pl
from jax.experimental.pallas import tpu as pltpu
from jax.experimental.pallas import tpu_sc as plsc

B = 4096
H = 50
D = 64
LS = 20
SEGS = ((0, 2), (2, 9), (9, 20))

NC = 2
NS = 16
NW = NC * NS
RPW = B // NW
NBUF = 2


def _sc_pool_kernel(table_hbm, lo_hbm, hi_hbm, out_hbm, lo_idx_v, hi_idx_v,
                    lo_b0, lo_b1, hi_b0, hi_b1, acc,
                    sem0, sem1, sem2, sem3):
    c = lax.axis_index("c")
    s = lax.axis_index("s")
    wid = s * NC + c
    base = wid * RPW

    lo_bufs = (lo_b0, lo_b1)
    hi_bufs = (hi_b0, hi_b1)
    lo_sems = (sem0, sem1)
    hi_sems = (sem2, sem3)

    pltpu.sync_copy(lo_hbm.at[wid], lo_idx_v)
    pltpu.sync_copy(hi_hbm.at[wid], hi_idx_v)

    def gather_lo(j, slot):
        return pltpu.make_async_copy(
            table_hbm.at[plsc.Indices(lo_idx_v.at[j], ignored_value=-1)],
            lo_bufs[slot], lo_sems[slot])

    def gather_hi(j, slot):
        return pltpu.make_async_copy(
            table_hbm.at[plsc.Indices(hi_idx_v.at[j], ignored_value=-1)],
            hi_bufs[slot], hi_sems[slot])

    z = jnp.zeros((16,), jnp.float32)

    def zero_all(r, _):
        for slot in range(NBUF):
            for d in range(4):
                lo_bufs[slot][r, pl.ds(d * 16, 16)] = z
                hi_bufs[slot][r, pl.ds(D + d * 16, 16)] = z
        for d in range(4):
            acc[r, pl.ds(d * 16, 16)] = z
        return 0
    lax.fori_loop(0, RPW, zero_all, 0)

    gather_lo(0, 0).start()
    gather_hi(0, 0).start()
    gather_lo(1, 1).start()
    gather_hi(1, 1).start()

    def accum(slot):
        lo_b, hi_b = lo_bufs[slot], hi_bufs[slot]

        def row_body(r, _):
            for d in range(4):
                lo = lo_b[r, pl.ds(d * 16, 16)]
                hi = hi_b[r, pl.ds(D + d * 16, 16)]
                plsc.addupdate(acc.at[r, pl.ds(d * 16, 16)], lo + hi)
                lo_b[r, pl.ds(d * 16, 16)] = z
                hi_b[r, pl.ds(D + d * 16, 16)] = z
            return 0
        lax.fori_loop(0, RPW, row_body, 0)

    def main_body(jj, _):
        for slot in range(NBUF):
            j = jj * NBUF + slot
            gather_lo(j, slot).wait()
            gather_hi(j, slot).wait()
            accum(slot)

            @pl.when(j + NBUF < H)
            def _():
                gather_lo(j + NBUF, slot).start()
                gather_hi(j + NBUF, slot).start()
        return 0

    lax.fori_loop(0, H // NBUF, main_body, 0)

    pltpu.sync_copy(acc, out_hbm.at[pl.ds(base, RPW)])


@functools.lru_cache(maxsize=None)
def _sc_pool():
    return pl.kernel(
        _sc_pool_kernel,
        out_type=jax.ShapeDtypeStruct((B, D), jnp.float32),
        mesh=plsc.VectorSubcoreMesh(core_axis_name="c", subcore_axis_name="s",
                                    num_cores=NC, num_subcores=NS),
        scratch_types=[
            pltpu.VMEM((H, RPW), jnp.int32),
            pltpu.VMEM((H, RPW), jnp.int32),
            pltpu.VMEM((RPW, 128), jnp.float32),
            pltpu.VMEM((RPW, 128), jnp.float32),
            pltpu.VMEM((RPW, 128), jnp.float32),
            pltpu.VMEM((RPW, 128), jnp.float32),
            pltpu.VMEM((RPW, D), jnp.float32),
            pltpu.SemaphoreType.DMA,
            pltpu.SemaphoreType.DMA,
            pltpu.SemaphoreType.DMA,
            pltpu.SemaphoreType.DMA,
        ],
    )


_PAD_BLK = 8192
_FOLD = 62 * _PAD_BLK


def _tc_pack_kernel(lo_ref, hi_ref, out_ref):
    out_ref[...] = jnp.concatenate([lo_ref[...].T, hi_ref[...].T], axis=1)


def _tc_pack(emb_t):
    return pl.pallas_call(
        _tc_pack_kernel,
        grid=(_FOLD // _PAD_BLK,),
        in_specs=[
            pl.BlockSpec((D, _PAD_BLK), lambda i: (0, i)),
            pl.BlockSpec((D, _PAD_BLK),
                         lambda i: (0, jnp.minimum(i, 60) + _FOLD // _PAD_BLK)),
        ],
        out_specs=pl.BlockSpec((_PAD_BLK, 2 * D), lambda i: (i, 0)),
        out_shape=jax.ShapeDtypeStruct((_FOLD, 2 * D), jnp.float32),
    )(emb_t, emb_t)


def _tc_head_kernel(us_ref, mask_ref, y_ref, ob_ref, w_ref,
                    logit_ref, loss_ref):
    x_len = jnp.sum(mask_ref[...], axis=1, keepdims=True)
    user_rep = us_ref[...] / x_len
    wu = lax.dot_general(user_rep, w_ref[...], (((1,), (1,)), ((), ())),
                         preferred_element_type=jnp.float32)
    y = y_ref[...]
    ob = ob_ref[...]
    col = lax.broadcasted_iota(jnp.int32, (1, LS), 1)
    loss = jnp.float32(0.0)
    logit = jnp.zeros((B, LS), jnp.float32)
    for (s, e) in SEGS:
        m = (col >= s) & (col < e)
        wc = jnp.where(m, wu * ob, 0.0)
        row_sum = jnp.sum(wc, axis=1)
        row_mask = (row_sum != 0.0).astype(jnp.float32)
        cnt = jnp.sum(row_mask)
        denom = jnp.sum(jnp.where(m, jnp.exp(wc), 0.0), axis=1)
        dot_y = jnp.sum(wc * y, axis=1)
        nll = jnp.sum(row_mask * (jnp.log(denom) - dot_y))
        loss = loss + jnp.where(cnt > 0, nll / cnt, jnp.float32(0.0))
        mx = jnp.max(jnp.where(m, wu, -1e30), axis=1, keepdims=True)
        ex = jnp.where(m, jnp.exp(wu - mx), 0.0)
        sm = ex / jnp.sum(ex, axis=1, keepdims=True)
        logit = jnp.where(m, sm, logit)
    logit_ref[...] = logit
    loss_ref[...] = jnp.broadcast_to(loss, (1, 1))


def _tc_head(user_sum, x_mask, y, ob, W):
    return pl.pallas_call(
        _tc_head_kernel,
        out_shape=(
            jax.ShapeDtypeStruct((B, LS), jnp.float32),
            jax.ShapeDtypeStruct((1, 1), jnp.float32),
        ),
    )(user_sum, x_mask, y, ob, W)


def kernel(x, x_mask, y, ob, item_emb, W):
    idx3 = x.reshape(NW, RPW, H).transpose(0, 2, 1)
    lo_idx3 = jnp.where(idx3 < _FOLD, idx3, -1)
    hi_idx3 = jnp.where(idx3 < _FOLD, -1, idx3 - _FOLD)
    table2 = _tc_pack(item_emb.T)
    user_sum = _sc_pool()(table2, lo_idx3, hi_idx3)
    logit, loss = _tc_head(user_sum, x_mask, y, ob, W)
    return (logit, loss[0, 0])


# speedup vs baseline: 1.1758x; 1.0434x over previous
import functools

import jax
import jax.numpy as jnp
from jax import lax
from jax.experimental import pallas as pl
from jax.experimental.pallas import tpu as pltpu
from jax.experimental.pallas import tpu_sc as plsc

B = 4096
H = 50
D = 64
LS = 20
SEGS = ((0, 2), (2, 9), (9, 20))

NC = 2
NS = 16
NW = NC * NS
RPW = B // NW
NBUF = 2


def _sc_pool_kernel(table_hbm, lo_hbm, hi_hbm, out_hbm, lo_idx_v, hi_idx_v,
                    lo_b0, lo_b1, hi_b0, hi_b1, acc,
                    sem0, sem1, sem2, sem3):
    c = lax.axis_index("c")
    s = lax.axis_index("s")
    wid = s * NC + c
    base = wid * RPW

    lo_bufs = (lo_b0, lo_b1)
    hi_bufs = (hi_b0, hi_b1)
    lo_sems = (sem0, sem1)
    hi_sems = (sem2, sem3)

    pltpu.sync_copy(lo_hbm.at[wid], lo_idx_v)
    pltpu.sync_copy(hi_hbm.at[wid], hi_idx_v)

    def gather_lo(j, slot):
        return pltpu.make_async_copy(
            table_hbm.at[plsc.Indices(lo_idx_v.at[j], ignored_value=-1)],
            lo_bufs[slot], lo_sems[slot])

    def gather_hi(j, slot):
        return pltpu.make_async_copy(
            table_hbm.at[plsc.Indices(hi_idx_v.at[j], ignored_value=-1)],
            hi_bufs[slot], hi_sems[slot])

    z = jnp.zeros((16,), jnp.float32)

    def zero_all(r, _):
        for slot in range(NBUF):
            for d in range(4):
                lo_bufs[slot][r, pl.ds(d * 16, 16)] = z
                hi_bufs[slot][r, pl.ds(D + d * 16, 16)] = z
        for d in range(4):
            acc[r, pl.ds(d * 16, 16)] = z
        return 0
    lax.fori_loop(0, RPW, zero_all, 0)

    gather_lo(0, 0).start()
    gather_hi(0, 0).start()
    gather_lo(1, 1).start()
    gather_hi(1, 1).start()

    def accum(slot):
        lo_b, hi_b = lo_bufs[slot], hi_bufs[slot]

        def row_body(r, _):
            for d in range(4):
                lo = lo_b[r, pl.ds(d * 16, 16)]
                hi = hi_b[r, pl.ds(D + d * 16, 16)]
                plsc.addupdate(acc.at[r, pl.ds(d * 16, 16)], lo + hi)
                lo_b[r, pl.ds(d * 16, 16)] = z
                hi_b[r, pl.ds(D + d * 16, 16)] = z
            return 0
        lax.fori_loop(0, RPW, row_body, 0)

    def main_body(jj, _):
        for slot in range(NBUF):
            j = jj * NBUF + slot
            gather_lo(j, slot).wait()
            gather_hi(j, slot).wait()
            accum(slot)

            @pl.when(j + NBUF < H)
            def _():
                gather_lo(j + NBUF, slot).start()
                gather_hi(j + NBUF, slot).start()
        return 0

    lax.fori_loop(0, H // NBUF, main_body, 0)

    pltpu.sync_copy(acc, out_hbm.at[pl.ds(base, RPW)])


@functools.lru_cache(maxsize=None)
def _sc_pool():
    return pl.kernel(
        _sc_pool_kernel,
        out_type=jax.ShapeDtypeStruct((B, D), jnp.float32),
        mesh=plsc.VectorSubcoreMesh(core_axis_name="c", subcore_axis_name="s",
                                    num_cores=NC, num_subcores=NS),
        scratch_types=[
            pltpu.VMEM((H, RPW), jnp.int32),
            pltpu.VMEM((H, RPW), jnp.int32),
            pltpu.VMEM((RPW, 128), jnp.float32),
            pltpu.VMEM((RPW, 128), jnp.float32),
            pltpu.VMEM((RPW, 128), jnp.float32),
            pltpu.VMEM((RPW, 128), jnp.float32),
            pltpu.VMEM((RPW, D), jnp.float32),
            pltpu.SemaphoreType.DMA,
            pltpu.SemaphoreType.DMA,
            pltpu.SemaphoreType.DMA,
            pltpu.SemaphoreType.DMA,
        ],
    )


_TABLE_N = 1000000
_PAD_BLK = 16384
_FOLD = 507904
_NBLK = _FOLD // _PAD_BLK
_LASTV = (_TABLE_N - 1) // _PAD_BLK


def _tc_pack_kernel(lo_ref, hi_ref, out_ref):
    out_ref[...] = jnp.concatenate([lo_ref[...].T, hi_ref[...].T], axis=1)


def _tc_pack(emb_t):
    return pl.pallas_call(
        _tc_pack_kernel,
        grid=(_NBLK,),
        in_specs=[
            pl.BlockSpec((D, _PAD_BLK), lambda i: (0, i)),
            pl.BlockSpec((D, _PAD_BLK),
                         lambda i: (0, jnp.minimum(i + _NBLK, _LASTV))),
        ],
        out_specs=pl.BlockSpec((_PAD_BLK, 2 * D), lambda i: (i, 0)),
        out_shape=jax.ShapeDtypeStruct((_FOLD, 2 * D), jnp.float32),
    )(emb_t, emb_t)


def _tc_head_kernel(us_ref, mask_ref, y_ref, ob_ref, w_ref,
                    logit_ref, loss_ref):
    x_len = jnp.sum(mask_ref[...], axis=1, keepdims=True)
    user_rep = us_ref[...] / x_len
    wu = lax.dot_general(user_rep, w_ref[...], (((1,), (1,)), ((), ())),
                         preferred_element_type=jnp.float32)
    y = y_ref[...]
    ob = ob_ref[...]
    col = lax.broadcasted_iota(jnp.int32, (1, LS), 1)
    loss = jnp.float32(0.0)
    logit = jnp.zeros((B, LS), jnp.float32)
    for (s, e) in SEGS:
        m = (col >= s) & (col < e)
        wc = jnp.where(m, wu * ob, 0.0)
        row_sum = jnp.sum(wc, axis=1)
        row_mask = (row_sum != 0.0).astype(jnp.float32)
        cnt = jnp.sum(row_mask)
        denom = jnp.sum(jnp.where(m, jnp.exp(wc), 0.0), axis=1)
        dot_y = jnp.sum(wc * y, axis=1)
        nll = jnp.sum(row_mask * (jnp.log(denom) - dot_y))
        loss = loss + jnp.where(cnt > 0, nll / cnt, jnp.float32(0.0))
        mx = jnp.max(jnp.where(m, wu, -1e30), axis=1, keepdims=True)
        ex = jnp.where(m, jnp.exp(wu - mx), 0.0)
        sm = ex / jnp.sum(ex, axis=1, keepdims=True)
        logit = jnp.where(m, sm, logit)
    logit_ref[...] = logit
    loss_ref[...] = jnp.broadcast_to(loss, (1, 1))


def _tc_head(user_sum, x_mask, y, ob, W):
    return pl.pallas_call(
        _tc_head_kernel,
        out_shape=(
            jax.ShapeDtypeStruct((B, LS), jnp.float32),
            jax.ShapeDtypeStruct((1, 1), jnp.float32),
        ),
    )(user_sum, x_mask, y, ob, W)


def kernel(x, x_mask, y, ob, item_emb, W):
    idx3 = x.reshape(NW, RPW, H).transpose(0, 2, 1)
    lo_idx3 = jnp.where(idx3 < _FOLD, idx3, -1)
    hi_idx3 = jnp.where(idx3 < _FOLD, -1, idx3 - _FOLD)
    table2 = _tc_pack(item_emb.T)
    user_sum = _sc_pool()(table2, lo_idx3, hi_idx3)
    logit, loss = _tc_head(user_sum, x_mask, y, ob, W)
    return (logit, loss[0, 0])
